# baseline (speedup 1.0000x reference)
import functools

import jax
import jax.numpy as jnp
from jax import lax
from jax.experimental import pallas as pl
from jax.experimental.pallas import tpu as pltpu
from jax.experimental.pallas import tpu_sc as plsc

NC, NS, LANES = 2, 16, 16
NW = NC * NS
CHUNK = 8
GDEPTH = 8
SDEPTH = 4


def kernel(state, timestep, embed_table):
    B, L, D = state.shape
    N = B * L
    state_f = state.reshape(N, D)
    ts_f = timestep.reshape(N)
    rows_per_w = N // NW
    n_chunks = rows_per_w // CHUNK
    n_groups = n_chunks // GDEPTH

    mesh = plsc.VectorSubcoreMesh(core_axis_name="c", subcore_axis_name="s")

    @functools.partial(
        pl.kernel,
        out_type=jax.ShapeDtypeStruct((N, D), jnp.float32),
        mesh=mesh,
        scratch_types=[
            pltpu.VMEM((rows_per_w,), jnp.int32),
            pltpu.VMEM((GDEPTH, CHUNK, D), jnp.float32),
            pltpu.VMEM((SDEPTH, CHUNK, D), jnp.float32),
        ] + [pltpu.SemaphoreType.DMA] * (GDEPTH + 2 * SDEPTH),
    )
    def sc_kernel(state_hbm, ts_hbm, table_hbm, out_hbm, idx_v, rows_v, so_v,
                  *sems):
        sem_g = sems[0:GDEPTH]
        sem_s = sems[GDEPTH:GDEPTH + SDEPTH]
        sem_o = sems[GDEPTH + SDEPTH:]
        wid = lax.axis_index("s") * NC + lax.axis_index("c")
        base_w = wid * rows_per_w

        pltpu.sync_copy(ts_hbm.at[pl.ds(base_w, rows_per_w)], idx_v)

        def issue_gather(ci, bg):
            pltpu.async_copy(table_hbm.at[idx_v.at[pl.ds(ci * CHUNK, CHUNK)]],
                             rows_v.at[bg], sem_g[bg])

        def wait_gather(ci, bg):
            pltpu.make_async_copy(
                table_hbm.at[idx_v.at[pl.ds(ci * CHUNK, CHUNK)]],
                rows_v.at[bg], sem_g[bg]).wait()

        def issue_state(ci, bs):
            pltpu.async_copy(
                state_hbm.at[pl.ds(base_w + ci * CHUNK, CHUNK), :],
                so_v.at[bs], sem_s[bs])

        def wait_state(ci, bs):
            pltpu.make_async_copy(
                state_hbm.at[pl.ds(base_w + ci * CHUNK, CHUNK), :],
                so_v.at[bs], sem_s[bs]).wait()

        def issue_out(ci, bs):
            pltpu.async_copy(so_v.at[bs],
                             out_hbm.at[pl.ds(base_w + ci * CHUNK, CHUNK), :],
                             sem_o[bs])

        def wait_out(ci, bs):
            pltpu.make_async_copy(
                so_v.at[bs],
                out_hbm.at[pl.ds(base_w + ci * CHUNK, CHUNK), :],
                sem_o[bs]).wait()

        def do_add(bg, bs):
            def row_body(r, carry):
                for j in range(D // LANES):
                    sl = pl.ds(j * LANES, LANES)
                    so_v[bs, r, sl] = so_v[bs, r, sl] + rows_v[bg, r, sl]
                return carry

            lax.fori_loop(0, CHUNK, row_body, 0)

        for k in range(GDEPTH):
            issue_gather(k, k)
        for k in range(SDEPTH - 1):
            issue_state(k, k)

        def group_body(g, carry):
            for k in range(GDEPTH):
                ci = g * GDEPTH + k
                bg = k
                bs = k % SDEPTH
                wait_gather(ci, bg)
                wait_state(ci, bs)
                do_add(bg, bs)
                issue_out(ci, bs)

                @pl.when(g < n_groups - 1)
                def _():
                    issue_gather(ci + GDEPTH, bg)

                if k == 0:
                    @pl.when(g > 0)
                    def _():
                        wait_out(ci - 1, (k - 1) % SDEPTH)
                else:
                    wait_out(ci - 1, (k - 1) % SDEPTH)

                @pl.when(ci + SDEPTH - 1 < n_chunks)
                def _():
                    issue_state(ci + SDEPTH - 1, (k - 1) % SDEPTH)

            return carry

        lax.fori_loop(0, n_groups, group_body, 0)

        wait_out(n_chunks - 1, (n_chunks - 1) % SDEPTH)

    out = sc_kernel(state_f, ts_f, embed_table)
    return out.reshape(B, L, D)
